# baseline (speedup 1.0000x reference)
import functools

import jax
import jax.numpy as jnp
from jax import lax
from jax.experimental import pallas as pl
from jax.experimental.pallas import tpu as pltpu
from jax.experimental.pallas import tpu_sc as plsc

S, H, I = 2048, 768, 384
E = 7
EP = 128
NEG = -1e30
TILE = 128
NA = 2 * S
NT_SH = S // TILE
NT_RT = NA // TILE + E
NT = NT_SH + NT_RT
N_XS = NT_RT * TILE
NB = S // EP

NC, NS = 2, 16
NW = NC * NS
TPW = S // NW


def _router_body(xr, wrr, rbr, sc_out, pm_out, tm_out):
    f32 = jnp.float32
    probs = jax.nn.sigmoid(xr[...] @ wrr[...] + rbr[...])
    lane = lax.broadcasted_iota(jnp.int32, (S, EP), 1)
    m0 = jnp.max(probs, axis=1, keepdims=True)
    i0 = jnp.min(jnp.where(probs == m0, lane, EP), axis=1, keepdims=True)
    probs1 = jnp.where(lane == i0, NEG, probs)
    m1 = jnp.max(probs1, axis=1, keepdims=True)
    i1 = jnp.min(jnp.where(probs1 == m1, lane, EP), axis=1, keepdims=True)
    lane32 = lax.broadcasted_iota(jnp.int32, (S, 32), 1)
    sc_out[...] = jnp.where(lane32 < 16, m0, m1)

    a0 = (lane == i0).astype(f32)
    a1 = (lane == i1).astype(f32)

    sub = lax.broadcasted_iota(jnp.int32, (EP, EP), 0)
    ln2 = lax.broadcasted_iota(jnp.int32, (EP, EP), 1)
    texcl = (ln2 < sub).astype(f32)
    ones_row = jnp.ones((1, EP), f32)
    mm = functools.partial(lax.dot, preferred_element_type=f32)

    def prefix(a, off0):
        off = off0
        parts = []
        for c in range(NB):
            blk = a[c * EP:(c + 1) * EP, :]
            parts.append(mm(texcl, blk) + off)
            off = off + mm(ones_row, blk)
        return jnp.concatenate(parts, axis=0), off

    zeros_row = jnp.zeros((1, EP), f32)
    r0, counts0 = prefix(a0, zeros_row)
    r1, counts = prefix(a1, counts0)
    tiles = jnp.floor((counts + (TILE - 1)) * (1.0 / TILE))
    cumt = mm(tiles, (sub <= ln2).astype(f32))
    slot_base = (cumt - tiles) * TILE + S

    pos0 = jnp.sum((r0 + slot_base) * a0, axis=1, keepdims=True)
    pos1 = jnp.sum((r1 + slot_base) * a1, axis=1, keepdims=True)
    lane8s = lax.broadcasted_iota(jnp.int32, (S, 8), 1)
    pm_out[...] = jnp.where(lane8s == 0, pos0.astype(jnp.int32),
                            jnp.where(lane8s == 1, pos1.astype(jnp.int32), 0))

    subc = lax.broadcasted_iota(jnp.int32, (EP, EP), 0)
    lnc = lax.broadcasted_iota(jnp.int32, (EP, EP), 1)
    jr = (subc - NT_SH).astype(f32)
    cumt_b = jnp.broadcast_to(cumt, (EP, EP))
    ind = ((cumt_b <= jr) & (lnc < E)).astype(f32)
    texp = jnp.sum(ind, axis=1, keepdims=True)
    nrt = jnp.sum(cumt * (lax.broadcasted_iota(jnp.int32, (1, EP), 1) == E - 1),
                  axis=1, keepdims=True)
    is_sh = subc[:, :1] < NT_SH
    jcol = (subc[:, :1] - NT_SH).astype(f32)
    texp_i = jnp.where(is_sh, E, jnp.clip(texp.astype(jnp.int32), 0, E - 1))
    act_i = jnp.where(is_sh | (jcol < jnp.broadcast_to(nrt, (EP, 1))), 1, 0)
    lane8t = lax.broadcasted_iota(jnp.int32, (EP, 8), 1)
    tm_out[...] = jnp.where(lane8t == 0, texp_i,
                            jnp.where(lane8t == 1, act_i, 0))


def _router_meta(xf, Wr, rbias):
    Wrp = jnp.zeros((H, EP), jnp.float32).at[:, :E].set(Wr)
    rbp = jnp.full((1, EP), NEG, jnp.float32).at[0, :E].set(rbias)
    return pl.pallas_call(
        _router_body,
        in_specs=[
            pl.BlockSpec((S, H), lambda: (0, 0)),
            pl.BlockSpec((H, EP), lambda: (0, 0)),
            pl.BlockSpec((1, EP), lambda: (0, 0)),
        ],
        out_specs=[
            pl.BlockSpec((S, 32), lambda: (0, 0)),
            pl.BlockSpec((S, 8), lambda: (0, 0)),
            pl.BlockSpec((EP, 8), lambda: (0, 0)),
        ],
        out_shape=[
            jax.ShapeDtypeStruct((S, 32), jnp.float32),
            jax.ShapeDtypeStruct((S, 8), jnp.int32),
            jax.ShapeDtypeStruct((EP, 8), jnp.int32),
        ],
    )(xf, Wrp, rbp)


def _dispatch_body(x_hbm, p0_hbm, p1_hbm, xs_hbm,
                   p0_v, p1_v, rows_v, sem0, sem1):
    wid = lax.axis_index("s") * NC + lax.axis_index("c")
    tb = wid * TPW
    pltpu.sync_copy(p0_hbm.at[pl.ds(tb, TPW)], p0_v)
    pltpu.sync_copy(p1_hbm.at[pl.ds(tb, TPW)], p1_v)
    pltpu.sync_copy(x_hbm.at[pl.ds(tb, TPW)], rows_v)
    c0 = pltpu.async_copy(rows_v, xs_hbm.at[p0_v], sem0)
    c1 = pltpu.async_copy(rows_v, xs_hbm.at[p1_v], sem1)
    pltpu.sync_copy(rows_v, xs_hbm.at[pl.ds(tb, TPW)])
    c0.wait()
    c1.wait()


def _dispatch(xf, p0, p1):
    mesh = plsc.VectorSubcoreMesh(core_axis_name="c", subcore_axis_name="s")
    k = pl.kernel(
        _dispatch_body,
        mesh=mesh,
        out_type=jax.ShapeDtypeStruct((NT * TILE, H), jnp.float32),
        scratch_types=[
            pltpu.VMEM((TPW,), jnp.int32),
            pltpu.VMEM((TPW,), jnp.int32),
            pltpu.VMEM((TPW, H), jnp.float32),
            pltpu.SemaphoreType.DMA,
            pltpu.SemaphoreType.DMA,
        ],
    )
    return k(xf, p0, p1)


def _gemm_body(te_ref, xsr, wgr, wur, wdr, wgsr, wusr, wdsr,
               ysr, wgb, wub, wdb):
    i = pl.program_id(0)
    bf = jnp.bfloat16
    te = te_ref[i, 0]
    mm = functools.partial(lax.dot, preferred_element_type=jnp.float32)

    @pl.when((i == 0) | (te != te_ref[jnp.maximum(i - 1, 0), 0]))
    def _():
        @pl.when(te == E)
        def _():
            wgb[...] = wgsr[...].astype(bf)
            wub[...] = wusr[...].astype(bf)
            wdb[...] = wdsr[...].astype(bf)

        @pl.when(te != E)
        def _():
            wgb[...] = wgr[0].astype(bf)
            wub[...] = wur[0].astype(bf)
            wdb[...] = wdr[0].astype(bf)

    src = xsr[...].astype(bf)
    h = jax.nn.silu(mm(src, wgb[...])) * mm(src, wub[...])
    ysr[...] = mm(h.astype(bf), wdb[...])


def _grouped_gemm(xs, Wg, Wu, Wd, Wg_s, Wu_s, Wd_s, tmap):
    grid_spec = pltpu.PrefetchScalarGridSpec(
        num_scalar_prefetch=1,
        grid=(NT,),
        in_specs=[
            pl.BlockSpec((TILE, H), lambda i, te: (i, 0)),
            pl.BlockSpec((1, H, I),
                         lambda i, te: (jnp.where(te[i, 0] == E, 0, te[i, 0]), 0, 0)),
            pl.BlockSpec((1, H, I),
                         lambda i, te: (jnp.where(te[i, 0] == E, 0, te[i, 0]), 0, 0)),
            pl.BlockSpec((1, I, H),
                         lambda i, te: (jnp.where(te[i, 0] == E, 0, te[i, 0]), 0, 0)),
            pl.BlockSpec((H, I), lambda i, te: (0, 0)),
            pl.BlockSpec((H, I), lambda i, te: (0, 0)),
            pl.BlockSpec((I, H), lambda i, te: (0, 0)),
        ],
        out_specs=pl.BlockSpec((TILE, H), lambda i, te: (i, 0)),
        scratch_shapes=[
            pltpu.VMEM((H, I), jnp.bfloat16),
            pltpu.VMEM((H, I), jnp.bfloat16),
            pltpu.VMEM((I, H), jnp.bfloat16),
        ],
    )
    return pl.pallas_call(
        _gemm_body,
        grid_spec=grid_spec,
        out_shape=jax.ShapeDtypeStruct((NT * TILE, H), jnp.float32),
        compiler_params=pltpu.CompilerParams(
            dimension_semantics=("arbitrary",),
        ),
    )(tmap, xs, Wg, Wu, Wd, Wg_s, Wu_s, Wd_s)


_CH = 32


def _combine_body(ys_hbm, p0_hbm, p1_hbm, sc_hbm, out_hbm,
                  acc_v, r0_v, r1_v, s_v, p0_v, p1_v, sem0, sem1):
    wid = lax.axis_index("s") * NC + lax.axis_index("c")
    for half in range(TPW // _CH):
        tb = wid * TPW + half * _CH
        pltpu.sync_copy(p0_hbm.at[pl.ds(tb, _CH)], p0_v)
        pltpu.sync_copy(p1_hbm.at[pl.ds(tb, _CH)], p1_v)
        g0 = pltpu.async_copy(ys_hbm.at[p0_v], r0_v, sem0)
        g1 = pltpu.async_copy(ys_hbm.at[p1_v], r1_v, sem1)
        pltpu.sync_copy(ys_hbm.at[pl.ds(tb, _CH)], acc_v)
        pltpu.sync_copy(sc_hbm.at[pl.ds(tb, _CH)], s_v)
        g0.wait()
        g1.wait()

        def body(j, _):
            s0 = s_v[j, pl.ds(0, 16)]
            s1 = s_v[j, pl.ds(16, 16)]
            for c in range(H // 16):
                sl = pl.ds(c * 16, 16)
                acc_v[j, sl] = acc_v[j, sl] + s0 * r0_v[j, sl] + s1 * r1_v[j, sl]
            return 0

        lax.fori_loop(0, _CH, body, 0)
        pltpu.sync_copy(acc_v, out_hbm.at[pl.ds(tb, _CH)])


def _combine(ys, p0, p1, scores):
    mesh = plsc.VectorSubcoreMesh(core_axis_name="c", subcore_axis_name="s")
    k = pl.kernel(
        _combine_body,
        mesh=mesh,
        out_type=jax.ShapeDtypeStruct((S, H), jnp.float32),
        scratch_types=[
            pltpu.VMEM((_CH, H), jnp.float32),
            pltpu.VMEM((_CH, H), jnp.float32),
            pltpu.VMEM((_CH, H), jnp.float32),
            pltpu.VMEM((_CH, 32), jnp.float32),
            pltpu.VMEM((_CH,), jnp.int32),
            pltpu.VMEM((_CH,), jnp.int32),
            pltpu.SemaphoreType.DMA,
            pltpu.SemaphoreType.DMA,
        ],
    )
    return k(ys, p0, p1, scores)


def kernel(x, Wg_s, Wu_s, Wd_s, Wg, Wu, Wd, Wr, rbias):
    xf = x.reshape(S, H)
    scores, pmat, tmap = _router_meta(xf, Wr, rbias)
    p0 = pmat[:, 0]
    p1 = pmat[:, 1]
    xs = _dispatch(xf, p0, p1)
    ys = _grouped_gemm(xs, Wg, Wu, Wd, Wg_s, Wu_s, Wd_s, tmap)
    out = _combine(ys, p0, p1, scores)
    return out.reshape(1, S, H)


# speedup vs baseline: 1.7748x; 1.7748x over previous
import jax
import jax.numpy as jnp
from jax.experimental import pallas as pl
from jax.experimental.pallas import tpu as pltpu

S, H, I = 2048, 768, 384
E = 7
EP = 128
NEG = -1e30
NSC = 4
SC = S // NSC


def _mm(a, b):
    return jax.lax.dot(a, b, preferred_element_type=jnp.float32)


def _dense_body(xr, wrr, rbr, wgsr, wusr, wdsr, wgr, wur, wdr, outr,
                wfull, wgb, wub, wdb):
    e = pl.program_id(0)
    s = pl.program_id(1)
    bf = jnp.bfloat16
    rows = pl.ds(s * SC, SC)
    xb = xr[rows, :].astype(bf)

    @pl.when(s == 0)
    def _():
        wgb[...] = wgr[0].astype(bf)
        wub[...] = wur[0].astype(bf)
        wdb[...] = wdr[0].astype(bf)

    @pl.when(e == 0)
    def _():
        h = jax.nn.silu(_mm(xb, wgsr[...].astype(bf))) * _mm(xb, wusr[...].astype(bf))
        outr[rows, :] = _mm(h.astype(bf), wdsr[...].astype(bf))
        probs = jax.nn.sigmoid(xr[rows, :] @ wrr[...] + rbr[...])
        lane = jax.lax.broadcasted_iota(jnp.int32, (SC, EP), 1)
        m0 = jnp.max(probs, axis=1, keepdims=True)
        i0 = jnp.min(jnp.where(probs == m0, lane, EP), axis=1, keepdims=True)
        probs1 = jnp.where(lane == i0, NEG, probs)
        m1 = jnp.max(probs1, axis=1, keepdims=True)
        i1 = jnp.min(jnp.where(probs1 == m1, lane, EP), axis=1, keepdims=True)
        wfull[rows, :] = m0 * (lane == i0) + m1 * (lane == i1)

    onehot = (jax.lax.broadcasted_iota(jnp.int32, (EP, 1), 0) == e).astype(jnp.float32)
    w_e = wfull[rows, :] @ onehot
    h = jax.nn.silu(_mm(xb, wgb[...])) * _mm(xb, wub[...])
    outr[rows, :] += _mm(h.astype(bf), wdb[...]) * w_e


def kernel(x, Wg_s, Wu_s, Wd_s, Wg, Wu, Wd, Wr, rbias):
    xf = x.reshape(S, H)
    Wrp = jnp.zeros((H, EP), jnp.float32).at[:, :E].set(Wr)
    rbp = jnp.full((1, EP), NEG, jnp.float32).at[0, :E].set(rbias)

    out = pl.pallas_call(
        _dense_body,
        grid=(E, NSC),
        in_specs=[
            pl.BlockSpec((S, H), lambda e, s: (0, 0)),
            pl.BlockSpec((H, EP), lambda e, s: (0, 0)),
            pl.BlockSpec((1, EP), lambda e, s: (0, 0)),
            pl.BlockSpec((H, I), lambda e, s: (0, 0)),
            pl.BlockSpec((H, I), lambda e, s: (0, 0)),
            pl.BlockSpec((I, H), lambda e, s: (0, 0)),
            pl.BlockSpec((1, H, I), lambda e, s: (e, 0, 0)),
            pl.BlockSpec((1, H, I), lambda e, s: (e, 0, 0)),
            pl.BlockSpec((1, I, H), lambda e, s: (e, 0, 0)),
        ],
        out_specs=pl.BlockSpec((S, H), lambda e, s: (0, 0)),
        out_shape=jax.ShapeDtypeStruct((S, H), jnp.float32),
        scratch_shapes=[
            pltpu.VMEM((S, EP), jnp.float32),
            pltpu.VMEM((H, I), jnp.bfloat16),
            pltpu.VMEM((H, I), jnp.bfloat16),
            pltpu.VMEM((I, H), jnp.bfloat16),
        ],
        compiler_params=pltpu.CompilerParams(
            dimension_semantics=("arbitrary", "arbitrary"),
        ),
    )(xf, Wrp, rbp, Wg_s, Wu_s, Wd_s, Wg, Wu, Wd)
    return out.reshape(1, S, H)


# speedup vs baseline: 2.1216x; 1.1954x over previous
import jax
import jax.numpy as jnp
from jax.experimental import pallas as pl
from jax.experimental.pallas import tpu as pltpu

S, H, I = 2048, 768, 384
E = 7
EP = 128
NEG = -1e30


def _mm(a, b):
    return jax.lax.dot(a, b, preferred_element_type=jnp.float32)


def _dense_body(xr, wrr, rbr, wgsr, wusr, wdsr, wgr, wur, wdr, outr, wfull, xbs):
    e = pl.program_id(0)
    bf = jnp.bfloat16

    @pl.when(e == 0)
    def _():
        xbs[...] = xr[...].astype(bf)

    xb = xbs[...]

    @pl.when(e == 0)
    def _():
        h = jax.nn.silu(_mm(xb, wgsr[...].astype(bf))) * _mm(xb, wusr[...].astype(bf))
        outr[...] = _mm(h.astype(bf), wdsr[...].astype(bf))
        probs = jax.nn.sigmoid(xr[...] @ wrr[...] + rbr[...])
        lane = jax.lax.broadcasted_iota(jnp.int32, (S, EP), 1)
        m0 = jnp.max(probs, axis=1, keepdims=True)
        i0 = jnp.min(jnp.where(probs == m0, lane, EP), axis=1, keepdims=True)
        probs1 = jnp.where(lane == i0, NEG, probs)
        m1 = jnp.max(probs1, axis=1, keepdims=True)
        i1 = jnp.min(jnp.where(probs1 == m1, lane, EP), axis=1, keepdims=True)
        wfull[...] = m0 * (lane == i0) + m1 * (lane == i1)

    onehot = (jax.lax.broadcasted_iota(jnp.int32, (EP, 1), 0) == e).astype(jnp.float32)
    w_e = wfull[...] @ onehot
    h = jax.nn.silu(_mm(xb, wgr[0].astype(bf))) * _mm(xb, wur[0].astype(bf))
    outr[...] += _mm(h.astype(bf), wdr[0].astype(bf)) * w_e


def kernel(x, Wg_s, Wu_s, Wd_s, Wg, Wu, Wd, Wr, rbias):
    xf = x.reshape(S, H)
    Wrp = jnp.zeros((H, EP), jnp.float32).at[:, :E].set(Wr)
    rbp = jnp.full((1, EP), NEG, jnp.float32).at[0, :E].set(rbias)

    out = pl.pallas_call(
        _dense_body,
        grid=(E,),
        in_specs=[
            pl.BlockSpec((S, H), lambda e: (0, 0)),
            pl.BlockSpec((H, EP), lambda e: (0, 0)),
            pl.BlockSpec((1, EP), lambda e: (0, 0)),
            pl.BlockSpec((H, I), lambda e: (0, 0)),
            pl.BlockSpec((H, I), lambda e: (0, 0)),
            pl.BlockSpec((I, H), lambda e: (0, 0)),
            pl.BlockSpec((1, H, I), lambda e: (e, 0, 0)),
            pl.BlockSpec((1, H, I), lambda e: (e, 0, 0)),
            pl.BlockSpec((1, I, H), lambda e: (e, 0, 0)),
        ],
        out_specs=pl.BlockSpec((S, H), lambda e: (0, 0)),
        out_shape=jax.ShapeDtypeStruct((S, H), jnp.float32),
        scratch_shapes=[
            pltpu.VMEM((S, EP), jnp.float32),
            pltpu.VMEM((S, H), jnp.bfloat16),
        ],
        compiler_params=pltpu.CompilerParams(
            dimension_semantics=("arbitrary",),
        ),
    )(xf, Wrp, rbp, Wg_s, Wu_s, Wd_s, Wg, Wu, Wd)
    return out.reshape(1, S, H)
